# baseline (speedup 1.0000x reference)
import functools
import math

import jax
import jax.numpy as jnp
from jax import lax
from jax.experimental import pallas as pl
from jax.experimental.pallas import tpu as pltpu
from jax.experimental.pallas import tpu_sc as plsc

VOCAB = 100000
D = 768
B = 4
S = 4096
N = B * S
SCALE = math.sqrt(float(D))

_info = plsc.get_sparse_core_info()
NC = _info.num_cores
NS = _info.num_subcores
NW = NC * NS
S_W = S // NW
R = 32
NCH = S_W // R
NU = NCH * B
LANES = 16
JV = D // LANES


def _sc_embed(idx_arr, table, pe_s):
    mesh = plsc.VectorSubcoreMesh(core_axis_name="c", subcore_axis_name="s")

    @functools.partial(
        pl.kernel,
        mesh=mesh,
        out_type=jax.ShapeDtypeStruct((N, D), jnp.float32),
        scratch_types=[
            pltpu.VMEM((NU, R), jnp.int32),
            pltpu.VMEM((2, R, D), jnp.float32),
            pltpu.VMEM((2, R, D), jnp.float32),
            pltpu.SemaphoreType.DMA,
            pltpu.SemaphoreType.DMA,
            pltpu.SemaphoreType.DMA,
            pltpu.SemaphoreType.DMA,
            pltpu.SemaphoreType.DMA,
            pltpu.SemaphoreType.DMA,
        ],
    )
    def k(idx_hbm, table_hbm, pe_hbm, out_hbm,
          idx_v, rows_v, pe_v, g0, g1, o0, o1, p0, p1):
        wid = lax.axis_index("s") * NC + lax.axis_index("c")
        sbase = wid * S_W
        g_sem = (g0, g1)
        o_sem = (o0, o1)
        p_sem = (p0, p1)

        def fire_gather(u, par):
            pltpu.async_copy(
                table_hbm.at[idx_v.at[u]], rows_v.at[par], g_sem[par])

        def drain_gather(u, par):
            pltpu.make_async_copy(
                table_hbm.at[idx_v.at[u]], rows_v.at[par], g_sem[par]).wait()

        def drain_out(par):
            pltpu.make_async_copy(
                rows_v.at[par], out_hbm.at[pl.ds(0, R)], o_sem[par]).wait()

        def fire_pe(sc, par):
            pltpu.async_copy(
                pe_hbm.at[pl.ds(sbase + sc * R, R)], pe_v.at[par], p_sem[par])

        def drain_pe(sc, par):
            pltpu.make_async_copy(
                pe_hbm.at[pl.ds(sbase + sc * R, R)],
                pe_v.at[par], p_sem[par]).wait()

        pltpu.sync_copy(idx_hbm.at[wid], idx_v)
        fire_pe(0, 0)
        fire_gather(0, 0)

        def unit(sc, b, pe_par, first, last_sc):
            par = b % 2
            nxt = 1 - par
            u = sc * B + b
            if b == 3:
                def pf():
                    drain_out(nxt)
                    fire_gather(u + 1, nxt)
                pl.when(jnp.logical_not(last_sc))(pf)
            else:
                def guard():
                    drain_out(nxt)
                pl.when(jnp.logical_not(first))(guard)
                fire_gather(u + 1, nxt)
            if b == 0:
                def pf_pe():
                    fire_pe(sc + 1, 1 - pe_par)
                pl.when(jnp.logical_not(last_sc))(pf_pe)
                drain_pe(sc, pe_par)
            drain_gather(u, par)

            def row(r, _):
                for j in range(JV):
                    sl = pl.ds(j * LANES, LANES)
                    rows_v[par, r, sl] = (
                        rows_v[par, r, sl] * SCALE + pe_v[pe_par, r, sl])
                return 0

            lax.fori_loop(0, R, row, 0)
            pltpu.async_copy(
                rows_v.at[par],
                out_hbm.at[pl.ds(b * S + sbase + sc * R, R)], o_sem[par])

        def group(g, _):
            sc0 = 2 * g
            for b in range(B):
                unit(sc0, b, 0, jnp.logical_and(g < 1, b < 1), jnp.bool_(False))
            for b in range(B):
                unit(sc0 + 1, b, 1, jnp.bool_(False), g >= NCH // 2 - 1)
            return 0

        lax.fori_loop(0, NCH // 2, group, 0)
        drain_out(0)
        drain_out(1)

    return k(idx_arr, table, pe_s)


def kernel(x, table, pe):
    idx_arr = (x.reshape(B, NW, NCH, R)
                .transpose(1, 2, 0, 3)
                .reshape(NW, NCH * B, R))
    out = _sc_embed(idx_arr, table, pe[:S])
    return out.reshape(B, S, D)


# speedup vs baseline: 1.1654x; 1.1654x over previous
import functools
import math

import jax
import jax.numpy as jnp
from jax import lax
from jax.experimental import pallas as pl
from jax.experimental.pallas import tpu as pltpu
from jax.experimental.pallas import tpu_sc as plsc

VOCAB = 100000
D = 768
B = 4
S = 4096
N = B * S
SCALE = math.sqrt(float(D))

_info = plsc.get_sparse_core_info()
NC = _info.num_cores
NS = _info.num_subcores
NW = NC * NS
S_W = S // NW
R = 32
NCH = S_W // R
LANES = 16
JV = D // LANES


def _sc_embed(idx_arr, table, pe_s):
    mesh = plsc.VectorSubcoreMesh(core_axis_name="c", subcore_axis_name="s")

    @functools.partial(
        pl.kernel,
        mesh=mesh,
        out_type=jax.ShapeDtypeStruct((N, D), jnp.float32),
        scratch_types=[
            pltpu.VMEM((NCH * B, R), jnp.int32),
            pltpu.VMEM((2, R, D), jnp.float32),
            pltpu.VMEM((R, D), jnp.float32),
            pltpu.SemaphoreType.DMA,
            pltpu.SemaphoreType.DMA,
            pltpu.SemaphoreType.DMA,
        ],
    )
    def k(idx_hbm, table_hbm, pe_hbm, out_hbm,
          idx_v, rows_v, pe_v, g_sem, o0, o1):
        wid = lax.axis_index("s") * NC + lax.axis_index("c")
        sbase = wid * S_W
        o_sem = (o0, o1)

        def drain_out(par):
            pltpu.make_async_copy(
                rows_v.at[par], out_hbm.at[pl.ds(0, R)], o_sem[par]).wait()

        pltpu.sync_copy(idx_hbm.at[wid], idx_v)

        def chunk(sc, _):
            pltpu.sync_copy(pe_hbm.at[pl.ds(sbase + sc * R, R)], pe_v)
            for b in range(B):
                par = b % 2
                if b < 2:
                    pl.when(sc >= 1)(lambda: drain_out(par))
                else:
                    drain_out(par)
                pltpu.async_copy(
                    table_hbm.at[idx_v.at[sc * B + b]],
                    rows_v.at[par], g_sem).wait()

                def row(r, _):
                    for j in range(JV):
                        sl = pl.ds(j * LANES, LANES)
                        rows_v[par, r, sl] = (
                            rows_v[par, r, sl] * SCALE + pe_v[r, sl])
                    return 0

                lax.fori_loop(0, R, row, 0)
                pltpu.async_copy(
                    rows_v.at[par],
                    out_hbm.at[pl.ds(b * S + sbase + sc * R, R)], o_sem[par])
            return 0

        lax.fori_loop(0, NCH, chunk, 0)
        drain_out(0)
        drain_out(1)

    return k(idx_arr, table, pe_s)


def kernel(x, table, pe):
    idx_arr = (x.reshape(B, NW, NCH, R)
                .transpose(1, 2, 0, 3)
                .reshape(NW, NCH * B, R))
    out = _sc_embed(idx_arr, table, pe[:S])
    return out.reshape(B, S, D)
